# Initial kernel scaffold; baseline (speedup 1.0000x reference)
import jax
import jax.numpy as jnp
from jax.experimental import pallas as pl


def kernel(atomic_numbers, per_system_total_charge, atomic_subsystem_indices, emb_table, W, b):
    raise NotImplementedError("write your pallas kernel here")


# speedup vs baseline: 4.2636x; 4.2636x over previous
import functools

import jax
import jax.numpy as jnp
from jax import lax
from jax.experimental import pallas as pl
from jax.experimental.pallas import tpu as pltpu
from jax.experimental.pallas import tpu_sc as plsc

N_ATOMS = 524288
N_MOL = 8192
FEAT = 64
MAX_Z = 100

NC = 2
NS = 16
NW = NC * NS
CHUNK = N_ATOMS // NW
BLK = 512
NBLK = CHUNK // BLK
GRP = BLK // 16


def _table_body(emb_ref, w_ref, b_ref, out_ref):
    w1 = w_ref[...][:, :FEAT]
    acc = lax.dot_general(
        emb_ref[...], w1, (((1,), (1,)), ((), ())),
        preferred_element_type=jnp.float32)
    out_ref[...] = acc + b_ref[...]


def _fused_table(emb, w, b2d):
    return pl.pallas_call(
        _table_body,
        out_shape=jax.ShapeDtypeStruct((MAX_Z, FEAT), jnp.float32),
    )(emb, w, b2d)


_MESH = plsc.VectorSubcoreMesh(
    core_axis_name="c", subcore_axis_name="s", num_cores=NC, num_subcores=NS)


@functools.partial(
    pl.kernel,
    out_type=jax.ShapeDtypeStruct((N_ATOMS * FEAT,), jnp.float32),
    mesh=_MESH,
    scratch_types=[
        pltpu.VMEM((MAX_Z * FEAT,), jnp.float32),
        pltpu.VMEM((N_MOL,), jnp.float32),
        pltpu.SMEM((FEAT,), jnp.float32),
        pltpu.VMEM_SHARED((FEAT,), jnp.float32),
        pltpu.VMEM((BLK,), jnp.int32),
        pltpu.VMEM((BLK,), jnp.int32),
        pltpu.VMEM((BLK,), jnp.int32),
        pltpu.VMEM((BLK,), jnp.int32),
        pltpu.VMEM((BLK * FEAT,), jnp.float32),
        pltpu.VMEM((BLK * FEAT,), jnp.float32),
        pltpu.SemaphoreType.DMA,
        pltpu.SemaphoreType.DMA,
        pltpu.SemaphoreType.DMA,
        pltpu.SemaphoreType.DMA,
    ],
    compiler_params=pltpu.CompilerParams(needs_layout_passes=False),
)
def _sc_featurize(tbl_hbm, w_hbm, z_hbm, s_hbm, chg_hbm, out_hbm,
                  tbl, chg, wsm, wvm, zb0, zb1, sb0, sb1, ob0, ob1,
                  semi0, semi1, semo0, semo1):
    zb = (zb0, zb1)
    sb = (sb0, sb1)
    ob = (ob0, ob1)
    semi = (semi0, semi1)
    semo = (semo0, semo1)
    wid = lax.axis_index("s") * NC + lax.axis_index("c")
    base = wid * CHUNK

    pltpu.sync_copy(tbl_hbm, tbl)
    pltpu.sync_copy(chg_hbm, chg)
    @pl.when(lax.axis_index("s") == 0)
    def _():
        pltpu.sync_copy(w_hbm, wvm)
    plsc.subcore_barrier()
    pltpu.sync_copy(wvm, wsm)

    iota64 = lax.iota(jnp.int32, 16) * FEAT

    pltpu.async_copy(z_hbm.at[pl.ds(base, BLK)], zb[0], semi[0])
    pltpu.async_copy(s_hbm.at[pl.ds(base, BLK)], sb[0], semi[0])

    @pl.loop(0, NBLK, step=2)
    def _blocks(blk2):
        for b in range(2):
            blk = blk2 + b
            rowbase = base + blk * BLK

            pltpu.make_async_copy(
                z_hbm.at[pl.ds(rowbase, BLK)], zb[b], semi[b]).wait()
            pltpu.make_async_copy(
                s_hbm.at[pl.ds(rowbase, BLK)], sb[b], semi[b]).wait()

            @pl.when(blk + 1 < NBLK)
            def _():
                nrow = rowbase + BLK
                pltpu.async_copy(
                    z_hbm.at[pl.ds(nrow, BLK)], zb[1 - b], semi[1 - b])
                pltpu.async_copy(
                    s_hbm.at[pl.ds(nrow, BLK)], sb[1 - b], semi[1 - b])

            @pl.when(blk >= 2)
            def _():
                prow = rowbase - 2 * BLK
                pltpu.make_async_copy(
                    ob[b],
                    out_hbm.at[pl.ds(prow * FEAT, BLK * FEAT)],
                    semo[b]).wait()

            obf = ob[b]

            @pl.loop(0, GRP)
            def _groups(g):
                z16 = zb[b][pl.ds(g * 16, 16)]
                s16 = sb[b][pl.ds(g * 16, 16)]
                c16 = plsc.load_gather(chg, [s16])
                z64 = z16 * FEAT
                oidx = iota64 + g * (16 * FEAT)

                @pl.loop(0, FEAT, unroll=8)
                def _feat(f):
                    t = plsc.load_gather(tbl, [z64 + f])
                    plsc.store_scatter(obf, [oidx + f], t + c16 * wsm[f])

            pltpu.async_copy(
                obf, out_hbm.at[pl.ds(rowbase * FEAT, BLK * FEAT)], semo[b])

    for b in range(2):
        tail = base + (NBLK - 2 + b) * BLK
        pltpu.make_async_copy(
            ob[b],
            out_hbm.at[pl.ds(tail * FEAT, BLK * FEAT)],
            semo[b]).wait()


def kernel(atomic_numbers, per_system_total_charge, atomic_subsystem_indices,
           emb_table, W, b):
    z = atomic_numbers.astype(jnp.int32)
    s = atomic_subsystem_indices.astype(jnp.int32)
    emb = emb_table.astype(jnp.float32)
    w = W.astype(jnp.float32)
    chg = per_system_total_charge.astype(jnp.float32)
    tbl = _fused_table(emb, w, b.astype(jnp.float32).reshape(1, FEAT))
    w_last = w[:, FEAT]
    out = _sc_featurize(tbl.reshape(-1), w_last, z, s, chg)
    return out.reshape(N_ATOMS, FEAT)


# speedup vs baseline: 10.4925x; 2.4610x over previous
import functools

import jax
import jax.numpy as jnp
from jax import lax
from jax.experimental import pallas as pl
from jax.experimental.pallas import tpu as pltpu
from jax.experimental.pallas import tpu_sc as plsc

N_ATOMS = 524288
N_MOL = 8192
FEAT = 64
MAX_Z = 100

NC = 2
NS = 16
NW = NC * NS
CHUNK = N_ATOMS // NW
BLK = 256
NBLK = CHUNK // BLK


def _table_body(emb_ref, w_ref, b_ref, out_ref):
    w1 = w_ref[...][:, :FEAT]
    acc = lax.dot_general(
        emb_ref[...], w1, (((1,), (1,)), ((), ())),
        preferred_element_type=jnp.float32)
    out_ref[...] = acc + b_ref[...]


def _fused_table(emb, w, b2d):
    return pl.pallas_call(
        _table_body,
        out_shape=jax.ShapeDtypeStruct((MAX_Z, FEAT), jnp.float32),
    )(emb, w, b2d)


_MESH = plsc.VectorSubcoreMesh(
    core_axis_name="c", subcore_axis_name="s", num_cores=NC, num_subcores=NS)


@functools.partial(
    pl.kernel,
    out_type=jax.ShapeDtypeStruct((N_ATOMS * FEAT,), jnp.float32),
    mesh=_MESH,
    scratch_types=[
        pltpu.VMEM((MAX_Z * FEAT,), jnp.float32),
        pltpu.VMEM((FEAT,), jnp.float32),
        pltpu.VMEM((BLK,), jnp.int32),
        pltpu.VMEM((BLK,), jnp.int32),
        pltpu.VMEM((BLK * FEAT,), jnp.float32),
        pltpu.VMEM((BLK * FEAT,), jnp.float32),
        pltpu.SMEM((BLK,), jnp.int32),
        pltpu.SMEM((BLK,), jnp.int32),
        pltpu.SMEM((BLK,), jnp.float32),
        pltpu.SMEM((BLK,), jnp.float32),
        pltpu.VMEM((BLK,), jnp.float32),
        pltpu.VMEM((BLK,), jnp.float32),
        pltpu.VMEM_SHARED((N_MOL,), jnp.float32),
        pltpu.VMEM_SHARED((NS * 2, BLK), jnp.int32),
        pltpu.VMEM_SHARED((NS * 2, BLK), jnp.float32),
        pltpu.SemaphoreType.DMA,
        pltpu.SemaphoreType.DMA,
        pltpu.SemaphoreType.DMA,
        pltpu.SemaphoreType.DMA,
        pltpu.SemaphoreType.DMA,
        pltpu.SemaphoreType.DMA,
    ],
    compiler_params=pltpu.CompilerParams(needs_layout_passes=False),
)
def _sc_featurize(tbl_hbm, w_hbm, z_hbm, s_hbm, chg_hbm, out_hbm,
                  tbl, wvm, sv0, sv1, ob0, ob1, zsm0, zsm1, csm0, csm1,
                  cv0, cv1, chg_sp, zsp, csp,
                  semA0, semA1, semB0, semB1, semo0, semo1):
    sv = (sv0, sv1)
    ob = (ob0, ob1)
    zsm = (zsm0, zsm1)
    csm = (csm0, csm1)
    cv = (cv0, cv1)
    semA = (semA0, semA1)
    semB = (semB0, semB1)
    semo = (semo0, semo1)

    tid = lax.axis_index("s")
    wid = tid * NC + lax.axis_index("c")
    base = wid * CHUNK

    pltpu.sync_copy(tbl_hbm, tbl)
    pltpu.sync_copy(w_hbm, wvm)

    @pl.when(tid == 0)
    def _():
        pltpu.sync_copy(chg_hbm, chg_sp)

    plsc.subcore_barrier()

    w4 = [wvm[pl.ds(16 * j, 16)] for j in range(4)]

    def issue_a(blk, b):
        row = base + blk * BLK
        pltpu.async_copy(z_hbm.at[pl.ds(row, BLK)], zsp.at[tid * 2 + b],
                         semA[b])
        pltpu.async_copy(s_hbm.at[pl.ds(row, BLK)], sv[b], semA[b])

    def wait_a(blk, b):
        row = base + blk * BLK
        pltpu.make_async_copy(z_hbm.at[pl.ds(row, BLK)],
                              zsp.at[tid * 2 + b], semA[b]).wait()
        pltpu.make_async_copy(s_hbm.at[pl.ds(row, BLK)], sv[b],
                              semA[b]).wait()

    def issue_b1(b):
        pltpu.async_copy(zsp.at[tid * 2 + b], zsm[b], semB[b])
        pltpu.async_copy(chg_sp.at[sv[b]], cv[b], semB[b])

    def wait_b1(b):
        pltpu.make_async_copy(zsp.at[tid * 2 + b], zsm[b], semB[b]).wait()
        pltpu.make_async_copy(chg_sp.at[sv[b]], cv[b], semB[b]).wait()

    def sync_b2(b):
        pltpu.sync_copy(cv[b], csp.at[tid * 2 + b])
        pltpu.sync_copy(csp.at[tid * 2 + b], csm[b])

    issue_a(0, 0)
    issue_a(1, 1)
    wait_a(0, 0)
    issue_b1(0)
    wait_b1(0)
    sync_b2(0)

    @pl.loop(0, NBLK, step=2)
    def _blocks(blk2):
        for b in range(2):
            blk = blk2 + b
            rowbase = base + blk * BLK

            @pl.when(blk + 2 < NBLK)
            def _():
                issue_a(blk + 2, b)

            @pl.when(blk + 1 < NBLK)
            def _():
                wait_a(blk + 1, 1 - b)
                issue_b1(1 - b)

            @pl.when(blk >= 2)
            def _():
                prow = rowbase - 2 * BLK
                pltpu.make_async_copy(
                    ob[b],
                    out_hbm.at[pl.ds(prow * FEAT, BLK * FEAT)],
                    semo[b]).wait()

            zsmb = zsm[b]
            csmb = csm[b]
            obf = ob[b]

            @pl.loop(0, BLK, unroll=4)
            def _atoms(a):
                zoff = zsmb[a] * FEAT
                c_a = csmb[a]
                arow = a * FEAT
                for j in range(4):
                    t = tbl[pl.ds(zoff + 16 * j, 16)]
                    obf[pl.ds(arow + 16 * j, 16)] = t + c_a * w4[j]

            pltpu.async_copy(
                obf, out_hbm.at[pl.ds(rowbase * FEAT, BLK * FEAT)], semo[b])

            @pl.when(blk + 1 < NBLK)
            def _():
                wait_b1(1 - b)
                sync_b2(1 - b)

    for b in range(2):
        tail = base + (NBLK - 2 + b) * BLK
        pltpu.make_async_copy(
            ob[b],
            out_hbm.at[pl.ds(tail * FEAT, BLK * FEAT)],
            semo[b]).wait()


def kernel(atomic_numbers, per_system_total_charge, atomic_subsystem_indices,
           emb_table, W, b):
    z = atomic_numbers.astype(jnp.int32)
    s = atomic_subsystem_indices.astype(jnp.int32)
    emb = emb_table.astype(jnp.float32)
    w = W.astype(jnp.float32)
    chg = per_system_total_charge.astype(jnp.float32)
    tbl = _fused_table(emb, w, b.astype(jnp.float32).reshape(1, FEAT))
    w_last = w[:, FEAT]
    out = _sc_featurize(tbl.reshape(-1), w_last, z, s, chg)
    return out.reshape(N_ATOMS, FEAT)


# speedup vs baseline: 13.1555x; 1.2538x over previous
import functools

import jax
import jax.numpy as jnp
from jax import lax
from jax.experimental import pallas as pl
from jax.experimental.pallas import tpu as pltpu
from jax.experimental.pallas import tpu_sc as plsc

N_ATOMS = 524288
N_MOL = 8192
FEAT = 64
MAX_Z = 100

NC = 2
NS = 16
NW = NC * NS
CHUNK = N_ATOMS // NW
BLK = 256
NBLK = CHUNK // BLK


def _table_body(emb_ref, w_ref, b_ref, out_ref):
    w1 = w_ref[...][:, :FEAT]
    acc = lax.dot_general(
        emb_ref[...], w1, (((1,), (1,)), ((), ())),
        preferred_element_type=jnp.float32)
    out_ref[...] = acc + b_ref[...]


def _fused_table(emb, w, b2d):
    return pl.pallas_call(
        _table_body,
        out_shape=jax.ShapeDtypeStruct((MAX_Z, FEAT), jnp.float32),
    )(emb, w, b2d)


_MESH = plsc.VectorSubcoreMesh(
    core_axis_name="c", subcore_axis_name="s", num_cores=NC, num_subcores=NS)


@functools.partial(
    pl.kernel,
    out_type=jax.ShapeDtypeStruct((N_ATOMS, FEAT), jnp.float32),
    mesh=_MESH,
    scratch_types=[
        pltpu.VMEM((MAX_Z * FEAT,), jnp.float32),
        pltpu.VMEM((FEAT,), jnp.float32),
        pltpu.VMEM((BLK,), jnp.int32),
        pltpu.VMEM((BLK,), jnp.int32),
        pltpu.VMEM((BLK, FEAT), jnp.float32),
        pltpu.VMEM((BLK, FEAT), jnp.float32),
        pltpu.SMEM((BLK,), jnp.int32),
        pltpu.SMEM((BLK,), jnp.int32),
        pltpu.SMEM((BLK,), jnp.float32),
        pltpu.SMEM((BLK,), jnp.float32),
        pltpu.VMEM((BLK,), jnp.float32),
        pltpu.VMEM((BLK,), jnp.float32),
        pltpu.VMEM_SHARED((N_MOL,), jnp.float32),
        pltpu.VMEM_SHARED((NS * 2, BLK), jnp.int32),
        pltpu.VMEM_SHARED((NS * 2, BLK), jnp.float32),
        pltpu.SemaphoreType.DMA,
        pltpu.SemaphoreType.DMA,
        pltpu.SemaphoreType.DMA,
        pltpu.SemaphoreType.DMA,
        pltpu.SemaphoreType.DMA,
        pltpu.SemaphoreType.DMA,
    ],
    compiler_params=pltpu.CompilerParams(needs_layout_passes=False),
)
def _sc_featurize(tbl_hbm, w_hbm, z_hbm, s_hbm, chg_hbm, out2d_hbm,
                  tbl, wvm, sv0, sv1, ob0, ob1, zsm0, zsm1, csm0, csm1,
                  cv0, cv1, chg_sp, zsp, csp,
                  semA0, semA1, semB0, semB1, semo0, semo1):
    sv = (sv0, sv1)
    ob = (ob0, ob1)
    zsm = (zsm0, zsm1)
    csm = (csm0, csm1)
    cv = (cv0, cv1)
    semA = (semA0, semA1)
    semB = (semB0, semB1)
    semo = (semo0, semo1)

    tid = lax.axis_index("s")
    wid = tid * NC + lax.axis_index("c")
    base = wid * CHUNK

    pltpu.sync_copy(tbl_hbm, tbl)
    pltpu.sync_copy(w_hbm, wvm)

    @pl.when(tid == 0)
    def _():
        pltpu.sync_copy(chg_hbm, chg_sp)

    plsc.subcore_barrier()

    w4 = [wvm[pl.ds(16 * j, 16)] for j in range(4)]

    def issue_a(blk, b):
        row = base + blk * BLK
        pltpu.async_copy(z_hbm.at[pl.ds(row, BLK)], zsp.at[tid * 2 + b],
                         semA[b])
        pltpu.async_copy(s_hbm.at[pl.ds(row, BLK)], sv[b], semA[b])

    def wait_a(blk, b):
        row = base + blk * BLK
        pltpu.make_async_copy(z_hbm.at[pl.ds(row, BLK)],
                              zsp.at[tid * 2 + b], semA[b]).wait()
        pltpu.make_async_copy(s_hbm.at[pl.ds(row, BLK)], sv[b],
                              semA[b]).wait()

    def issue_b1(b):
        pltpu.async_copy(zsp.at[tid * 2 + b], zsm[b], semB[b])
        pltpu.async_copy(chg_sp.at[sv[b]], cv[b], semB[b])

    def wait_b1(b):
        pltpu.make_async_copy(zsp.at[tid * 2 + b], zsm[b], semB[b]).wait()
        pltpu.make_async_copy(chg_sp.at[sv[b]], cv[b], semB[b]).wait()

    def sync_b2(b):
        pltpu.sync_copy(cv[b], csp.at[tid * 2 + b])
        pltpu.sync_copy(csp.at[tid * 2 + b], csm[b])

    issue_a(0, 0)
    issue_a(1, 1)
    wait_a(0, 0)
    issue_b1(0)
    wait_b1(0)
    sync_b2(0)

    @pl.loop(0, NBLK, step=2)
    def _blocks(blk2):
        for b in range(2):
            blk = blk2 + b
            rowbase = base + blk * BLK

            @pl.when(blk + 2 < NBLK)
            def _():
                issue_a(blk + 2, b)

            @pl.when(blk + 1 < NBLK)
            def _():
                wait_a(blk + 1, 1 - b)
                issue_b1(1 - b)

            @pl.when(blk >= 2)
            def _():
                prow = rowbase - 2 * BLK
                pltpu.make_async_copy(
                    ob[b],
                    out2d_hbm.at[pl.ds(prow, BLK)],
                    semo[b]).wait()

            zsmb = zsm[b]
            csmb = csm[b]
            obf = ob[b]

            @pl.loop(0, BLK, unroll=4)
            def _atoms(a):
                zoff = zsmb[a] * FEAT
                c_a = csmb[a]
                for j in range(4):
                    t = tbl[pl.ds(zoff + 16 * j, 16)]
                    obf[a, pl.ds(16 * j, 16)] = t + c_a * w4[j]

            pltpu.async_copy(
                obf, out2d_hbm.at[pl.ds(rowbase, BLK)], semo[b])

            @pl.when(blk + 1 < NBLK)
            def _():
                wait_b1(1 - b)
                sync_b2(1 - b)

    for b in range(2):
        tail = base + (NBLK - 2 + b) * BLK
        pltpu.make_async_copy(
            ob[b],
            out2d_hbm.at[pl.ds(tail, BLK)],
            semo[b]).wait()


def kernel(atomic_numbers, per_system_total_charge, atomic_subsystem_indices,
           emb_table, W, b):
    z = atomic_numbers.astype(jnp.int32)
    s = atomic_subsystem_indices.astype(jnp.int32)
    emb = emb_table.astype(jnp.float32)
    w = W.astype(jnp.float32)
    chg = per_system_total_charge.astype(jnp.float32)
    tbl = _fused_table(emb, w, b.astype(jnp.float32).reshape(1, FEAT))
    w_last = w[:, FEAT]
    return _sc_featurize(tbl.reshape(-1), w_last, z, s, chg)
